# baseline (speedup 1.0000x reference)
import functools

import jax
import jax.numpy as jnp
from jax import lax
from jax.experimental import pallas as pl
from jax.experimental.pallas import tpu as pltpu
from jax.experimental.pallas import tpu_sc as plsc

VOCAB = 1000
VPAD = 1024
NSUB = VPAD // 128
NC, NS = 2, 16
NW = NC * NS
B, T = 1024, 50
NBB = B // 128
UNITS = T * NSUB * NBB
U_PER_W = UNITS // NW


def _make_kernel():
  mesh = plsc.VectorSubcoreMesh(core_axis_name="c", subcore_axis_name="s",
                                num_cores=NC, num_subcores=NS)

  @functools.partial(
      pl.kernel,
      mesh=mesh,
      out_type=jax.ShapeDtypeStruct((T, VOCAB, B), jnp.float32),
      scratch_types=[
          pltpu.VMEM((U_PER_W, 128), jnp.int32),
          pltpu.VMEM((128, 128), jnp.float32),
          pltpu.VMEM((128, 128), jnp.float32),
          pltpu.VMEM((128, 128), jnp.float32),
          pltpu.SemaphoreType.DMA,
          pltpu.SemaphoreType.DMA,
      ],
      compiler_params=pltpu.CompilerParams(use_tc_tiling_on_sc=True,
                                           needs_layout_passes=False),
  )
  def gather_kernel(table_hbm, idx_hbm, out_hbm, idx_v, gbuf0, gbuf1, tbuf,
                    sem0, sem1):
    wid = lax.axis_index("s") * NC + lax.axis_index("c")
    pltpu.sync_copy(idx_hbm.at[wid], idx_v)
    lanes = lax.broadcasted_iota(jnp.int32, (16,), 0)

    def gather(k, buf, sem):
      return pltpu.make_async_copy(table_hbm.at[idx_v.at[k]], buf, sem)

    def emit(k, buf):
      def body_b(b, carry):
        cols = jnp.full((16,), 0, jnp.int32) + b
        for vg in range(NSUB):
          val = buf[b, pl.ds(vg * 16, 16)]
          plsc.store_scatter(tbuf, [lanes + (vg * 16), cols], val)
        return carry

      lax.fori_loop(0, 128, body_b, 0)
      u = wid * U_PER_W + k
      t = u // (NSUB * NBB)
      r = u % (NSUB * NBB)
      j = r // NBB
      b0 = (r % NBB) * 128
      @pl.when(j < NSUB - 1)
      def _():
        pltpu.sync_copy(
            tbuf, out_hbm.at[t, pl.ds(j * 128, 128), pl.ds(b0, 128)])

      @pl.when(j == NSUB - 1)
      def _():
        pltpu.sync_copy(
            tbuf.at[pl.ds(0, VOCAB - (NSUB - 1) * 128)],
            out_hbm.at[t, pl.ds((NSUB - 1) * 128, VOCAB - (NSUB - 1) * 128),
                       pl.ds(b0, 128)])

    gather(0, gbuf0, sem0).start()

    def body(i, carry):
      k0 = 2 * i
      k1 = k0 + 1
      gather(k1, gbuf1, sem1).start()
      gather(k0, gbuf0, sem0).wait()
      emit(k0, gbuf0)

      @pl.when(i < U_PER_W // 2 - 1)
      def _():
        gather(k1 + 1, gbuf0, sem0).start()

      gather(k1, gbuf1, sem1).wait()
      emit(k1, gbuf1)
      return carry

    lax.fori_loop(0, U_PER_W // 2, body, 0)

  return gather_kernel


_sc_gather = _make_kernel()


def _impl(idx, table):
  idx32 = idx.astype(jnp.int32)
  sub = idx32.T[:, None, :] * NSUB + jnp.arange(NSUB, dtype=jnp.int32)[None, :, None]
  sub = sub.reshape(T, NSUB, NBB, 128).reshape(NW, U_PER_W, 128)
  table2 = jnp.pad(table, ((0, 0), (0, VPAD - VOCAB))).reshape(VOCAB * NSUB, 128)
  out = _sc_gather(table2, sub)
  return jnp.transpose(out, (2, 0, 1))


kernel = jax.jit(_impl)


# speedup vs baseline: 1.0722x; 1.0722x over previous
import functools

import jax
import jax.numpy as jnp
from jax import lax
from jax.experimental import pallas as pl
from jax.experimental.pallas import tpu as pltpu
from jax.experimental.pallas import tpu_sc as plsc

VOCAB = 1000
VPAD = 1024
NSUB = VPAD // 128
NC, NS = 2, 16
NW = NC * NS
B, T = 1024, 50
NBB = B // 128
UNITS = T * NSUB * NBB
U_PER_W = UNITS // NW
VLAST = VOCAB - (NSUB - 1) * 128


def _make_kernel():
  mesh = plsc.VectorSubcoreMesh(core_axis_name="c", subcore_axis_name="s",
                                num_cores=NC, num_subcores=NS)

  @functools.partial(
      pl.kernel,
      mesh=mesh,
      out_type=jax.ShapeDtypeStruct((T, VOCAB, B), jnp.float32),
      scratch_types=[
          pltpu.VMEM((U_PER_W, 128), jnp.int32),
          pltpu.VMEM((128, 128), jnp.float32),
          pltpu.VMEM((128, 128), jnp.float32),
          pltpu.VMEM((128, 128), jnp.float32),
          pltpu.VMEM((128, 128), jnp.float32),
          pltpu.SemaphoreType.DMA,
          pltpu.SemaphoreType.DMA,
          pltpu.SemaphoreType.DMA,
          pltpu.SemaphoreType.DMA,
      ],
      compiler_params=pltpu.CompilerParams(use_tc_tiling_on_sc=True,
                                           needs_layout_passes=False),
  )
  def gather_kernel(table_hbm, idx_hbm, out_hbm, idx_v, gbuf0, gbuf1, tbuf0,
                    tbuf1, gsem0, gsem1, dsem0, dsem1):
    wid = lax.axis_index("s") * NC + lax.axis_index("c")
    pltpu.sync_copy(idx_hbm.at[wid], idx_v)
    lanes = lax.broadcasted_iota(jnp.int32, (16,), 0)
    rowidx = [lanes + vg * 16 for vg in range(NSUB)]

    def gather(k, buf, sem):
      return pltpu.make_async_copy(table_hbm.at[idx_v.at[k]], buf, sem)

    def transpose(buf, tbuf):
      def body_b(i, cols):
        b = 2 * i
        cols1 = cols + 1
        vals0 = [buf[b, pl.ds(vg * 16, 16)] for vg in range(NSUB)]
        vals1 = [buf[b + 1, pl.ds(vg * 16, 16)] for vg in range(NSUB)]
        for vg in range(NSUB):
          plsc.store_scatter(tbuf, [rowidx[vg], cols], vals0[vg])
        for vg in range(NSUB):
          plsc.store_scatter(tbuf, [rowidx[vg], cols1], vals1[vg])
        return cols + 2

      lax.fori_loop(0, 64, body_b, jnp.zeros((16,), jnp.int32), unroll=False)

    def drain(k, tbuf, sem):
      u = wid * U_PER_W + k
      t = u // (NSUB * NBB)
      r = u % (NSUB * NBB)
      j = r // NBB
      b0 = (r % NBB) * 128

      def mk_full():
        return pltpu.make_async_copy(
            tbuf, out_hbm.at[t, pl.ds(j * 128, 128), pl.ds(b0, 128)], sem)

      def mk_last():
        return pltpu.make_async_copy(
            tbuf.at[pl.ds(0, VLAST)],
            out_hbm.at[t, pl.ds((NSUB - 1) * 128, VLAST), pl.ds(b0, 128)],
            sem)

      return j, mk_full, mk_last

    def drain_start(k, tbuf, sem):
      j, mk_full, mk_last = drain(k, tbuf, sem)

      @pl.when(j < NSUB - 1)
      def _():
        mk_full().start()

      @pl.when(j == NSUB - 1)
      def _():
        mk_last().start()

    def drain_wait(k, tbuf, sem):
      j, mk_full, mk_last = drain(k, tbuf, sem)

      @pl.when(j < NSUB - 1)
      def _():
        mk_full().wait()

      @pl.when(j == NSUB - 1)
      def _():
        mk_last().wait()

    gather(0, gbuf0, gsem0).start()
    gather(1, gbuf1, gsem1).start()

    def step(i, k, gbuf, gsem, tbuf, dsem):
      gather(k, gbuf, gsem).wait()

      @pl.when(i > 0)
      def _():
        drain_wait(k - 2, tbuf, dsem)

      transpose(gbuf, tbuf)
      drain_start(k, tbuf, dsem)

      @pl.when(k + 2 < U_PER_W)
      def _():
        gather(k + 2, gbuf, gsem).start()

    def body(i, carry):
      step(i, 2 * i, gbuf0, gsem0, tbuf0, dsem0)
      step(i, 2 * i + 1, gbuf1, gsem1, tbuf1, dsem1)
      return carry

    lax.fori_loop(0, U_PER_W // 2, body, 0)
    drain_wait(U_PER_W - 2, tbuf0, dsem0)
    drain_wait(U_PER_W - 1, tbuf1, dsem1)

  return gather_kernel


_sc_gather = _make_kernel()


def _impl(idx, table):
  idx32 = idx.astype(jnp.int32)
  sub = idx32.T[:, None, :] * NSUB + jnp.arange(NSUB, dtype=jnp.int32)[None, :, None]
  sub = sub.reshape(T, NSUB, NBB, 128).reshape(NW, U_PER_W, 128)
  table2 = jnp.pad(table, ((0, 0), (0, VPAD - VOCAB))).reshape(VOCAB * NSUB, 128)
  out = _sc_gather(table2, sub)
  return jnp.transpose(out, (2, 0, 1))


kernel = jax.jit(_impl)


# speedup vs baseline: 2.4455x; 2.2808x over previous
import functools

import jax
import jax.numpy as jnp
from jax import lax
from jax.experimental import pallas as pl
from jax.experimental.pallas import tpu as pltpu
from jax.experimental.pallas import tpu_sc as plsc

VOCAB = 1000
VPAD = 1024
NSUB = VPAD // 128
NC, NS = 2, 16
NW = NC * NS
B, T = 1024, 50
NBB = B // 128
UNITS = T * NSUB * NBB
U_PER_W = UNITS // NW
VLAST = VOCAB - (NSUB - 1) * 128


def _make_kernel():
  mesh = plsc.VectorSubcoreMesh(core_axis_name="c", subcore_axis_name="s",
                                num_cores=NC, num_subcores=NS)

  @functools.partial(
      pl.kernel,
      mesh=mesh,
      out_type=jax.ShapeDtypeStruct((T, VOCAB, B), jnp.float32),
      scratch_types=[
          pltpu.VMEM((U_PER_W, 128), jnp.int32),
          pltpu.VMEM((128, 128), jnp.float32),
          pltpu.VMEM((128, 128), jnp.float32),
          pltpu.VMEM((128, 128), jnp.float32),
          pltpu.VMEM((128, 128), jnp.float32),
          pltpu.SemaphoreType.DMA,
          pltpu.SemaphoreType.DMA,
          pltpu.SemaphoreType.DMA,
          pltpu.SemaphoreType.DMA,
      ],
      compiler_params=pltpu.CompilerParams(use_tc_tiling_on_sc=True,
                                           needs_layout_passes=False),
  )
  def gather_kernel(table_hbm, idx_hbm, out_hbm, idx_v, gbuf0, gbuf1, tbuf0,
                    tbuf1, gsem0, gsem1, dsem0, dsem1):
    wid = lax.axis_index("s") * NC + lax.axis_index("c")
    pltpu.sync_copy(idx_hbm.at[wid], idx_v)
    lanes = lax.broadcasted_iota(jnp.int32, (16,), 0)
    pats = [(lanes + d) & 15 for d in range(16)]

    def gather(k, buf, sem):
      return pltpu.make_async_copy(table_hbm.at[idx_v.at[k]], buf, sem)

    def transpose(buf, tbuf):
      def body_tile(tt, carry):
        v0 = (tt // 8) * 16
        b0 = (tt % 8) * 16
        rows_b = lanes + b0
        for d in range(16):
          cols_v = pats[d] + v0
          val = plsc.load_gather(buf, [rows_b, cols_v])
          plsc.store_scatter(tbuf, [cols_v, rows_b], val)
        return carry

      lax.fori_loop(0, 64, body_tile, 0, unroll=False)

    def drain(k, tbuf, sem):
      u = wid * U_PER_W + k
      t = u // (NSUB * NBB)
      r = u % (NSUB * NBB)
      j = r // NBB
      b0 = (r % NBB) * 128

      def mk_full():
        return pltpu.make_async_copy(
            tbuf, out_hbm.at[t, pl.ds(j * 128, 128), pl.ds(b0, 128)], sem)

      def mk_last():
        return pltpu.make_async_copy(
            tbuf.at[pl.ds(0, VLAST)],
            out_hbm.at[t, pl.ds((NSUB - 1) * 128, VLAST), pl.ds(b0, 128)],
            sem)

      return j, mk_full, mk_last

    def drain_start(k, tbuf, sem):
      j, mk_full, mk_last = drain(k, tbuf, sem)

      @pl.when(j < NSUB - 1)
      def _():
        mk_full().start()

      @pl.when(j == NSUB - 1)
      def _():
        mk_last().start()

    def drain_wait(k, tbuf, sem):
      j, mk_full, mk_last = drain(k, tbuf, sem)

      @pl.when(j < NSUB - 1)
      def _():
        mk_full().wait()

      @pl.when(j == NSUB - 1)
      def _():
        mk_last().wait()

    gather(0, gbuf0, gsem0).start()
    gather(1, gbuf1, gsem1).start()

    def step(i, k, gbuf, gsem, tbuf, dsem):
      gather(k, gbuf, gsem).wait()

      @pl.when(i > 0)
      def _():
        drain_wait(k - 2, tbuf, dsem)

      transpose(gbuf, tbuf)
      drain_start(k, tbuf, dsem)

      @pl.when(k + 2 < U_PER_W)
      def _():
        gather(k + 2, gbuf, gsem).start()

    def body(i, carry):
      step(i, 2 * i, gbuf0, gsem0, tbuf0, dsem0)
      step(i, 2 * i + 1, gbuf1, gsem1, tbuf1, dsem1)
      return carry

    lax.fori_loop(0, U_PER_W // 2, body, 0)
    drain_wait(U_PER_W - 2, tbuf0, dsem0)
    drain_wait(U_PER_W - 1, tbuf1, dsem1)

  return gather_kernel


_sc_gather = _make_kernel()


def _impl(idx, table):
  idx32 = idx.astype(jnp.int32)
  sub = idx32.T[:, None, :] * NSUB + jnp.arange(NSUB, dtype=jnp.int32)[None, :, None]
  sub = sub.reshape(T, NSUB, NBB, 128).reshape(NW, U_PER_W, 128)
  table2 = jnp.pad(table, ((0, 0), (0, VPAD - VOCAB))).reshape(VOCAB * NSUB, 128)
  out = _sc_gather(table2, sub)
  return jnp.transpose(out, (2, 0, 1))


kernel = jax.jit(_impl)


# speedup vs baseline: 4.1101x; 1.6807x over previous
import functools

import jax
import jax.numpy as jnp
from jax import lax
from jax.experimental import pallas as pl
from jax.experimental.pallas import tpu as pltpu
from jax.experimental.pallas import tpu_sc as plsc

VOCAB = 1000
VPAD = 1024
NSUB = VPAD // 128
NC, NS = 2, 16
NW = NC * NS
B, T = 1024, 50
NBB = B // 128
UNITS = T * NSUB * NBB
U_PER_W = UNITS // NW
VLAST = VOCAB - (NSUB - 1) * 128


def _make_kernel():
  mesh = plsc.VectorSubcoreMesh(core_axis_name="c", subcore_axis_name="s",
                                num_cores=NC, num_subcores=NS)

  @functools.partial(
      pl.kernel,
      mesh=mesh,
      out_type=jax.ShapeDtypeStruct((T, VOCAB, B), jnp.float32),
      scratch_types=[
          pltpu.VMEM((U_PER_W, 128), jnp.int32),
          pltpu.VMEM((128, 128), jnp.float32),
          pltpu.VMEM((128, 128), jnp.float32),
          pltpu.VMEM((128 * 128,), jnp.float32),
          pltpu.VMEM((128, 128), jnp.float32),
          pltpu.VMEM((128, 128), jnp.float32),
          pltpu.SemaphoreType.DMA,
          pltpu.SemaphoreType.DMA,
          pltpu.SemaphoreType.DMA,
          pltpu.SemaphoreType.DMA,
      ],
      compiler_params=pltpu.CompilerParams(use_tc_tiling_on_sc=True,
                                           needs_layout_passes=False),
  )
  def gather_kernel(table_hbm, idx_hbm, out_hbm, idx_v, gbuf0, gbuf1, sbuf,
                    tbuf0, tbuf1, gsem0, gsem1, dsem0, dsem1):
    wid = lax.axis_index("s") * NC + lax.axis_index("c")
    pltpu.sync_copy(idx_hbm.at[wid], idx_v)
    lanes = lax.broadcasted_iota(jnp.int32, (16,), 0)
    rowidx = [lanes + vg * 16 for vg in range(NSUB)]
    flatbase = [(lanes + vg * 16) * 128 for vg in range(NSUB)]

    def gather(k, buf, sem):
      return pltpu.make_async_copy(table_hbm.at[idx_v.at[k]], buf, sem)

    def skew_pass(buf):
      def body_b(b, cols_b):
        vals = [buf[b, pl.ds(vg * 16, 16)] for vg in range(NSUB)]
        for vg in range(NSUB):
          idxs = flatbase[vg] + ((cols_b + rowidx[vg]) & 127)
          plsc.store_scatter(sbuf, [idxs], vals[vg])
        return cols_b + 1

      lax.fori_loop(0, 128, body_b, jnp.zeros((16,), jnp.int32), unroll=False)

    def unskew_pass(tbuf):
      def body_v(v, carry):
        rows_v, w = carry
        vals = [sbuf[pl.ds(v * 128 + q * 16, 16)] for q in range(NSUB)]
        for q in range(NSUB):
          plsc.store_scatter(tbuf, [rows_v, (w + q * 16) & 127], vals[q])
        return rows_v + 1, w - 1

      lax.fori_loop(0, 128, body_v, (jnp.zeros((16,), jnp.int32), lanes),
                    unroll=False)

    def drain(k, tbuf, sem):
      u = wid * U_PER_W + k
      t = u // (NSUB * NBB)
      r = u % (NSUB * NBB)
      j = r // NBB
      b0 = (r % NBB) * 128

      def mk_full():
        return pltpu.make_async_copy(
            tbuf, out_hbm.at[t, pl.ds(j * 128, 128), pl.ds(b0, 128)], sem)

      def mk_last():
        return pltpu.make_async_copy(
            tbuf.at[pl.ds(0, VLAST)],
            out_hbm.at[t, pl.ds((NSUB - 1) * 128, VLAST), pl.ds(b0, 128)],
            sem)

      return j, mk_full, mk_last

    def drain_start(k, tbuf, sem):
      j, mk_full, mk_last = drain(k, tbuf, sem)

      @pl.when(j < NSUB - 1)
      def _():
        mk_full().start()

      @pl.when(j == NSUB - 1)
      def _():
        mk_last().start()

    def drain_wait(k, tbuf, sem):
      j, mk_full, mk_last = drain(k, tbuf, sem)

      @pl.when(j < NSUB - 1)
      def _():
        mk_full().wait()

      @pl.when(j == NSUB - 1)
      def _():
        mk_last().wait()

    gather(0, gbuf0, gsem0).start()
    gather(1, gbuf1, gsem1).start()

    def step(i, k, gbuf, gsem, tbuf, dsem):
      gather(k, gbuf, gsem).wait()
      skew_pass(gbuf)

      @pl.when(k + 2 < U_PER_W)
      def _():
        gather(k + 2, gbuf, gsem).start()

      @pl.when(i > 0)
      def _():
        drain_wait(k - 2, tbuf, dsem)

      unskew_pass(tbuf)
      drain_start(k, tbuf, dsem)

    def body(i, carry):
      step(i, 2 * i, gbuf0, gsem0, tbuf0, dsem0)
      step(i, 2 * i + 1, gbuf1, gsem1, tbuf1, dsem1)
      return carry

    lax.fori_loop(0, U_PER_W // 2, body, 0)
    drain_wait(U_PER_W - 2, tbuf0, dsem0)
    drain_wait(U_PER_W - 1, tbuf1, dsem1)

  return gather_kernel


_sc_gather = _make_kernel()


def _impl(idx, table):
  idx32 = idx.astype(jnp.int32)
  sub = idx32.T[:, None, :] * NSUB + jnp.arange(NSUB, dtype=jnp.int32)[None, :, None]
  sub = sub.reshape(T, NSUB, NBB, 128).reshape(NW, U_PER_W, 128)
  table2 = jnp.pad(table, ((0, 0), (0, VPAD - VOCAB))).reshape(VOCAB * NSUB, 128)
  out = _sc_gather(table2, sub)
  return jnp.transpose(out, (2, 0, 1))


kernel = jax.jit(_impl)


# speedup vs baseline: 4.1123x; 1.0005x over previous
import functools

import jax
import jax.numpy as jnp
from jax import lax
from jax.experimental import pallas as pl
from jax.experimental.pallas import tpu as pltpu
from jax.experimental.pallas import tpu_sc as plsc

VOCAB = 1000
VPAD = 1024
NSUB = VPAD // 128
NC, NS = 2, 16
NW = NC * NS
B, T = 1024, 50
NBB = B // 128
UNITS = T * NSUB * NBB
U_PER_W = UNITS // NW
VLAST = VOCAB - (NSUB - 1) * 128


def _make_kernel():
  mesh = plsc.VectorSubcoreMesh(core_axis_name="c", subcore_axis_name="s",
                                num_cores=NC, num_subcores=NS)

  @functools.partial(
      pl.kernel,
      mesh=mesh,
      out_type=jax.ShapeDtypeStruct((T, VOCAB, B), jnp.float32),
      scratch_types=[
          pltpu.VMEM((U_PER_W, 128), jnp.int32),
          pltpu.VMEM((128, 128), jnp.float32),
          pltpu.VMEM((128, 128), jnp.float32),
          pltpu.VMEM((128 * 128,), jnp.float32),
          pltpu.VMEM((128, 128), jnp.float32),
          pltpu.VMEM((128, 128), jnp.float32),
          pltpu.SemaphoreType.DMA,
          pltpu.SemaphoreType.DMA,
          pltpu.SemaphoreType.DMA,
          pltpu.SemaphoreType.DMA,
      ],
      compiler_params=pltpu.CompilerParams(use_tc_tiling_on_sc=True,
                                           needs_layout_passes=False),
  )
  def gather_kernel(table_hbm, idx_hbm, out_hbm, idx_v, gbuf0, gbuf1, sbuf,
                    tbuf0, tbuf1, gsem0, gsem1, dsem0, dsem1):
    wid = lax.axis_index("s") * NC + lax.axis_index("c")
    pltpu.sync_copy(idx_hbm.at[wid], idx_v)
    lanes = lax.broadcasted_iota(jnp.int32, (16,), 0)
    rowidx = [lanes + vg * 16 for vg in range(NSUB)]
    flatbase = [(lanes + vg * 16) * 128 for vg in range(NSUB)]

    def gather(k, buf, sem):
      return pltpu.make_async_copy(table_hbm.at[idx_v.at[k]], buf, sem)

    def skew_pass(buf):
      def body_b(b, cols_b):
        vals = [buf[b, pl.ds(vg * 16, 16)] for vg in range(NSUB)]
        for vg in range(NSUB):
          idxs = flatbase[vg] + ((cols_b + rowidx[vg]) & 127)
          plsc.store_scatter(sbuf, [idxs], vals[vg])
        return cols_b + 1

      lax.fori_loop(0, 128, body_b, jnp.zeros((16,), jnp.int32), unroll=4)

    def unskew_pass(tbuf):
      def body_v(v, carry):
        rows_v, w = carry
        vals = [sbuf[pl.ds(v * 128 + q * 16, 16)] for q in range(NSUB)]
        for q in range(NSUB):
          plsc.store_scatter(tbuf, [rows_v, (w + q * 16) & 127], vals[q])
        return rows_v + 1, w - 1

      lax.fori_loop(0, 128, body_v, (jnp.zeros((16,), jnp.int32), lanes),
                    unroll=4)

    def drain(k, tbuf, sem):
      u = wid * U_PER_W + k
      t = u // (NSUB * NBB)
      r = u % (NSUB * NBB)
      j = r // NBB
      b0 = (r % NBB) * 128

      def mk_full():
        return pltpu.make_async_copy(
            tbuf, out_hbm.at[t, pl.ds(j * 128, 128), pl.ds(b0, 128)], sem)

      def mk_last():
        return pltpu.make_async_copy(
            tbuf.at[pl.ds(0, VLAST)],
            out_hbm.at[t, pl.ds((NSUB - 1) * 128, VLAST), pl.ds(b0, 128)],
            sem)

      return j, mk_full, mk_last

    def drain_start(k, tbuf, sem):
      j, mk_full, mk_last = drain(k, tbuf, sem)

      @pl.when(j < NSUB - 1)
      def _():
        mk_full().start()

      @pl.when(j == NSUB - 1)
      def _():
        mk_last().start()

    def drain_wait(k, tbuf, sem):
      j, mk_full, mk_last = drain(k, tbuf, sem)

      @pl.when(j < NSUB - 1)
      def _():
        mk_full().wait()

      @pl.when(j == NSUB - 1)
      def _():
        mk_last().wait()

    gather(0, gbuf0, gsem0).start()
    gather(1, gbuf1, gsem1).start()

    def step(i, k, gbuf, gsem, tbuf, dsem):
      gather(k, gbuf, gsem).wait()
      skew_pass(gbuf)

      @pl.when(k + 2 < U_PER_W)
      def _():
        gather(k + 2, gbuf, gsem).start()

      @pl.when(i > 0)
      def _():
        drain_wait(k - 2, tbuf, dsem)

      unskew_pass(tbuf)
      drain_start(k, tbuf, dsem)

    def body(i, carry):
      step(i, 2 * i, gbuf0, gsem0, tbuf0, dsem0)
      step(i, 2 * i + 1, gbuf1, gsem1, tbuf1, dsem1)
      return carry

    lax.fori_loop(0, U_PER_W // 2, body, 0)
    drain_wait(U_PER_W - 2, tbuf0, dsem0)
    drain_wait(U_PER_W - 1, tbuf1, dsem1)

  return gather_kernel


_sc_gather = _make_kernel()


def _impl(idx, table):
  idx32 = idx.astype(jnp.int32)
  sub = idx32.T[:, None, :] * NSUB + jnp.arange(NSUB, dtype=jnp.int32)[None, :, None]
  sub = sub.reshape(T, NSUB, NBB, 128).reshape(NW, U_PER_W, 128)
  table2 = jnp.pad(table, ((0, 0), (0, VPAD - VOCAB))).reshape(VOCAB * NSUB, 128)
  out = _sc_gather(table2, sub)
  return jnp.transpose(out, (2, 0, 1))


kernel = jax.jit(_impl)
